# baseline (speedup 1.0000x reference)
import functools

import jax
import jax.numpy as jnp
from jax import lax
from jax.experimental import pallas as pl
from jax.experimental.pallas import tpu as pltpu
from jax.experimental.pallas import tpu_sc as plsc

B = 4096
L = 200
D = 64
H = 32
NC = 2
NS = 16
NW = NC * NS
BPW = B // NW
LA = 128
LB = L - LA
NV = D // 16


def _pool_sc(x, emb):
    mesh = plsc.VectorSubcoreMesh(core_axis_name="core", subcore_axis_name="subcore")

    @functools.partial(
        pl.kernel,
        out_type=jax.ShapeDtypeStruct((B, D), jnp.float32),
        mesh=mesh,
        scratch_types=[
            pltpu.VMEM((BPW, L), jnp.int32),
            pltpu.VMEM((LA, D), jnp.float32),
            pltpu.VMEM((LB, D), jnp.float32),
            pltpu.VMEM((BPW, D), jnp.float32),
            pltpu.SemaphoreType.DMA,
            pltpu.SemaphoreType.DMA,
        ],
        compiler_params=pltpu.CompilerParams(use_tc_tiling_on_sc=False),
    )
    def pool(x_hbm, emb_hbm, out_hbm, idx_v, rows_a, rows_b, acc_v, sem_a, sem_b):
        wid = lax.axis_index("subcore") * NC + lax.axis_index("core")
        base = wid * BPW
        pltpu.sync_copy(x_hbm.at[pl.ds(base, BPW)], idx_v)

        @pl.loop(0, BPW)
        def _(b):
            ca = pltpu.async_copy(emb_hbm.at[idx_v.at[b, pl.ds(0, LA)]], rows_a, sem_a)
            cb = pltpu.async_copy(emb_hbm.at[idx_v.at[b, pl.ds(LA, LB)]], rows_b, sem_b)
            ca.wait()
            cb.wait()

            def body_a(r, accs):
                return tuple(a + rows_a[r, pl.ds(16 * i, 16)] for i, a in enumerate(accs))

            accs = lax.fori_loop(
                0, LA, body_a, tuple(jnp.zeros((16,), jnp.float32) for _ in range(NV))
            )

            def body_b(r, accs):
                return tuple(a + rows_b[r, pl.ds(16 * i, 16)] for i, a in enumerate(accs))

            accs = lax.fori_loop(0, LB, body_b, accs)
            for i in range(NV):
                acc_v[b, pl.ds(16 * i, 16)] = accs[i]

        pltpu.sync_copy(acc_v, out_hbm.at[pl.ds(base, BPW)])

    return pool(x, emb)


def _mlp_tc(pooled_sum, w1t, b1, w2, b2):
    def body(p_ref, w1_ref, b1_ref, w2_ref, b2_ref, o_ref):
        p = p_ref[...] * (1.0 / L)
        h = jnp.dot(p, w1_ref[...], preferred_element_type=jnp.float32) + b1_ref[...]
        h = jnp.maximum(h, 0.0)
        o_ref[...] = jnp.sum(h * w2_ref[...], axis=1, keepdims=True) + b2_ref[...]

    return pl.pallas_call(
        body,
        out_shape=jax.ShapeDtypeStruct((B, 1), jnp.float32),
    )(pooled_sum, w1t, b1, w2, b2)


def kernel(x, emb, W1, b1, W2, b2):
    pooled_sum = _pool_sc(x, emb)
    out = _mlp_tc(
        pooled_sum,
        W1.T,
        b1.reshape(1, H),
        W2.reshape(1, H),
        b2.reshape(1, 1),
    )
    return out.reshape(B)


# speedup vs baseline: 1.1866x; 1.1866x over previous
import functools

import jax
import jax.numpy as jnp
from jax import lax
from jax.experimental import pallas as pl
from jax.experimental.pallas import tpu as pltpu
from jax.experimental.pallas import tpu_sc as plsc

B = 4096
L = 200
D = 64
H = 32
NC = 2
NS = 16
NW = NC * NS
BPW = B // NW
NACC = 4
NV = D // 16


def _pool_sc(x_t, emb):
    mesh = plsc.VectorSubcoreMesh(core_axis_name="core", subcore_axis_name="subcore")

    @functools.partial(
        pl.kernel,
        out_type=jax.ShapeDtypeStruct((B, D), jnp.float32),
        mesh=mesh,
        scratch_types=[
            pltpu.VMEM((L, BPW), jnp.int32),
            pltpu.VMEM((NACC, BPW, D), jnp.float32),
            pltpu.VMEM((BPW, D), jnp.float32),
        ]
        + [pltpu.SemaphoreType.DMA] * NACC,
        compiler_params=pltpu.CompilerParams(use_tc_tiling_on_sc=False),
    )
    def pool(xt_hbm, emb_hbm, out_hbm, idx_v, accs_v, out_v, *sems):
        wid = lax.axis_index("subcore") * NC + lax.axis_index("core")
        base = wid * BPW
        pltpu.sync_copy(xt_hbm.at[:, pl.ds(base, BPW)], idx_v)

        for k in range(NACC):
            pltpu.async_copy(emb_hbm.at[idx_v.at[k]], accs_v.at[k], sems[k])

        @pl.loop(NACC, L, step=NACC)
        def _(j):
            for k in range(NACC):
                pltpu.make_async_copy(
                    emb_hbm.at[idx_v.at[0]], accs_v.at[k], sems[k]
                ).wait()
                pltpu.async_copy(
                    emb_hbm.at[idx_v.at[j + k]], accs_v.at[k], sems[k], add=True
                )

        for k in range(NACC):
            pltpu.make_async_copy(emb_hbm.at[idx_v.at[0]], accs_v.at[k], sems[k]).wait()

        @pl.loop(0, BPW)
        def _(b):
            for i in range(NV):
                s = pl.ds(16 * i, 16)
                out_v[b, s] = (
                    (accs_v[0, b, s] + accs_v[1, b, s])
                    + (accs_v[2, b, s] + accs_v[3, b, s])
                )

        pltpu.sync_copy(out_v, out_hbm.at[pl.ds(base, BPW)])

    return pool(x_t, emb)


def _mlp_tc(pooled_sum, w1t, b1, w2, b2):
    def body(p_ref, w1_ref, b1_ref, w2_ref, b2_ref, o_ref):
        p = p_ref[...] * (1.0 / L)
        h = jnp.dot(p, w1_ref[...], preferred_element_type=jnp.float32) + b1_ref[...]
        h = jnp.maximum(h, 0.0)
        o_ref[...] = jnp.sum(h * w2_ref[...], axis=1, keepdims=True) + b2_ref[...]

    return pl.pallas_call(
        body,
        out_shape=jax.ShapeDtypeStruct((B, 1), jnp.float32),
    )(pooled_sum, w1t, b1, w2, b2)


def kernel(x, emb, W1, b1, W2, b2):
    pooled_sum = _pool_sc(x.T, emb)
    out = _mlp_tc(
        pooled_sum,
        W1.T,
        b1.reshape(1, H),
        W2.reshape(1, H),
        b2.reshape(1, 1),
    )
    return out.reshape(B)
